# baseline (speedup 1.0000x reference)
import functools

import jax
import jax.numpy as jnp
from jax import lax
from jax.experimental import pallas as pl
from jax.experimental.pallas import tpu as pltpu
from jax.experimental.pallas import tpu_sc as plsc

_D = 64
_B = 4096
_QT = 20
_DT = 200
_BQ = _B * _QT
_BD = _B * _DT
_NC = 2
_NS = 16
_NW = _NC * _NS
_CH = 256
_QW = _BQ // _NW
_DW = _BD // _NW
_QC = _QW // _CH
_DC = _DW // _CH
_CPB = _B // _CH

_mesh = plsc.VectorSubcoreMesh(
    core_axis_name="c", subcore_axis_name="s", num_cores=_NC, num_subcores=_NS
)


@functools.partial(
    pl.kernel,
    out_type=(
        jax.ShapeDtypeStruct((_QT, 8, 32, 8, 128), jnp.float32),
        jax.ShapeDtypeStruct((_DT, 8, 32, 8, 128), jnp.float32),
    ),
    mesh=_mesh,
    compiler_params=pltpu.CompilerParams(
        use_tc_tiling_on_sc=False, needs_layout_passes=False),
    scratch_types=[
        pltpu.VMEM((_QW,), jnp.int32),
        pltpu.VMEM((_DW,), jnp.int32),
        pltpu.VMEM((_CH, _D), jnp.float32),
        pltpu.VMEM((_CH, _D), jnp.float32),
        pltpu.VMEM((8, 2, 8, 128), jnp.float32),
        pltpu.VMEM((8, 2, 8, 128), jnp.float32),
        pltpu.SemaphoreType.DMA,
        pltpu.SemaphoreType.DMA,
        pltpu.SemaphoreType.DMA,
        pltpu.SemaphoreType.DMA,
    ],
)
def _embed_gather(q_hbm, d_hbm, table_hbm, qo_hbm, do_hbm,
                  qidx, didx, r0, r1, t0, t1, gs0, gs1, ws0, ws1):
    w = lax.axis_index("s") * _NC + lax.axis_index("c")

    pltpu.sync_copy(q_hbm.at[pl.ds(w * _QW, _QW)], qidx)
    pltpu.sync_copy(d_hbm.at[pl.ds(w * _DW, _DW)], didx)

    def _shift(idx_ref, n):
        def body(i, carry):
            idx_ref[pl.ds(i * 16, 16)] = idx_ref[pl.ds(i * 16, 16)] + 1
            return carry
        lax.fori_loop(0, n // 16, body, 0)

    _shift(qidx, _QW)
    _shift(didx, _DW)

    riota = lax.iota(jnp.int32, 16)
    row16 = [[riota + (cc * 128 + l0 * 16) for l0 in range(8)] for cc in range(2)]

    def fire_gather(idx_ref, j, rbuf, sem):
        pltpu.async_copy(
            table_hbm.at[idx_ref.at[pl.ds(j * _CH, 128)]],
            rbuf.at[pl.ds(0, 128)], sem)
        pltpu.async_copy(
            table_hbm.at[idx_ref.at[pl.ds(j * _CH + 128, 128)]],
            rbuf.at[pl.ds(128, 128)], sem)

    def drain_gather(rbuf, sem):
        pltpu.make_async_copy(
            table_hbm.at[pl.ds(0, 128)], rbuf.at[pl.ds(0, 128)], sem).wait()
        pltpu.make_async_copy(
            table_hbm.at[pl.ds(0, 128)], rbuf.at[pl.ds(128, 128)], sem).wait()

    def transpose(rbuf, tbuf):
        def body(gr, carry):
            g = gr // 8
            r = gr - g * 8
            col16 = jnp.full((16,), 8 * g + r, jnp.int32)
            for cc in range(2):
                for l0 in range(8):
                    v = plsc.load_gather(rbuf, [row16[cc][l0], col16])
                    tbuf[g, cc, r, pl.ds(l0 * 16, 16)] = v
            return carry
        lax.fori_loop(0, 64, body, 0)

    def fire_writes(tbuf, out_hbm, t, c0, sem):
        for g in range(8):
            pltpu.async_copy(tbuf.at[g], out_hbm.at[t, g, pl.ds(c0, 2)], sem)

    def drain_writes(tbuf, out_hbm, sem):
        for g in range(8):
            pltpu.make_async_copy(
                out_hbm.at[0, 0, pl.ds(0, 2)], tbuf.at[g], sem).wait()

    def run(idx_ref, nchunks, out_hbm, base_chunk):
        def coords(j):
            J = base_chunk + j
            t = J // _CPB
            c0 = (J - t * _CPB) * 2
            return t, c0

        fire_gather(idx_ref, 0, r0, gs0)

        def body(jj, carry):
            ja = 2 * jj
            jb = ja + 1
            fire_gather(idx_ref, jb, r1, gs1)
            drain_gather(r0, gs0)
            transpose(r0, t0)
            ta, ca = coords(ja)
            fire_writes(t0, out_hbm, ta, ca, ws0)

            @pl.when(jj < nchunks // 2 - 1)
            def _():
                fire_gather(idx_ref, ja + 2, r0, gs0)

            drain_gather(r1, gs1)
            transpose(r1, t1)
            tb, cb = coords(jb)
            fire_writes(t1, out_hbm, tb, cb, ws1)
            drain_writes(t0, out_hbm, ws0)
            drain_writes(t1, out_hbm, ws1)
            return carry

        lax.fori_loop(0, nchunks // 2, body, 0)

    run(qidx, _QC, qo_hbm, w * _QC)
    run(didx, _DC, do_hbm, w * _DC)


def kernel(query_tok, doc_tok, table):
    q_idx = query_tok.T.reshape(_BQ).astype(jnp.int32)
    d_idx = doc_tok.T.reshape(_BD).astype(jnp.int32)
    q5, d5 = _embed_gather(q_idx, d_idx, table)
    q_emb = q5.transpose(2, 4, 0, 1, 3).reshape(_B, _QT, _D)
    d_emb = d5.transpose(2, 4, 0, 1, 3).reshape(_B, _DT, _D)
    return (q_emb, d_emb)


# speedup vs baseline: 1.4819x; 1.4819x over previous
import functools

import jax
import jax.numpy as jnp
from jax import lax
from jax.experimental import pallas as pl
from jax.experimental.pallas import tpu as pltpu
from jax.experimental.pallas import tpu_sc as plsc

_D = 64
_B = 4096
_QT = 20
_DT = 200
_NC = 2
_NS = 16
_NW = _NC * _NS
_CH = 256
_CPR = _B // _CH

_mesh = plsc.VectorSubcoreMesh(
    core_axis_name="c", subcore_axis_name="s", num_cores=_NC, num_subcores=_NS
)


def _make_gather(total_t, base_t, num_t):
    nchunks = num_t * _CPR
    cpw = nchunks // _NW
    span = (cpw - 1) // _CPR + 2
    span = min(span, num_t)

    @functools.partial(
        pl.kernel,
        out_type=jax.ShapeDtypeStruct((num_t * _B, _D), jnp.float32),
        mesh=_mesh,
        compiler_params=pltpu.CompilerParams(
            use_tc_tiling_on_sc=False, needs_layout_passes=False),
        scratch_types=[
            pltpu.VMEM((span, _B), jnp.int32),
            pltpu.VMEM((_CH, _D), jnp.float32),
            pltpu.VMEM((_CH, _D), jnp.float32),
            pltpu.SemaphoreType.DMA,
            pltpu.SemaphoreType.DMA,
            pltpu.SemaphoreType.DMA,
            pltpu.SemaphoreType.DMA,
        ],
    )
    def gather(tok_hbm, table_hbm, out_hbm, idx, r0, r1, gs0, gs1, ws0, ws1):
        w = lax.axis_index("s") * _NC + lax.axis_index("c")
        j0 = w * cpw
        t0 = jnp.minimum(j0 // _CPR, num_t - span)

        pltpu.sync_copy(tok_hbm.at[pl.ds(base_t + t0, span)], idx)

        def shift(i, carry):
            r = i // (_B // 16)
            o = (i - r * (_B // 16)) * 16
            idx[r, pl.ds(o, 16)] = idx[r, pl.ds(o, 16)] + 1
            return carry
        lax.fori_loop(0, span * (_B // 16), shift, 0)

        def fire_gather(j, rbuf, sem):
            t = j // _CPR
            off = (j - t * _CPR) * _CH
            lr = t - t0
            pltpu.async_copy(
                table_hbm.at[idx.at[lr, pl.ds(off, 128)]],
                rbuf.at[pl.ds(0, 128)], sem)
            pltpu.async_copy(
                table_hbm.at[idx.at[lr, pl.ds(off + 128, 128)]],
                rbuf.at[pl.ds(128, 128)], sem)

        def drain_gather(rbuf, sem):
            pltpu.make_async_copy(
                table_hbm.at[pl.ds(0, 128)], rbuf.at[pl.ds(0, 128)], sem).wait()
            pltpu.make_async_copy(
                table_hbm.at[pl.ds(0, 128)], rbuf.at[pl.ds(128, 128)], sem).wait()

        def fire_write(j, rbuf, sem):
            pltpu.async_copy(rbuf, out_hbm.at[pl.ds(j * _CH, _CH)], sem)

        def drain_write(rbuf, sem):
            pltpu.make_async_copy(
                out_hbm.at[pl.ds(0, _CH)], rbuf, sem).wait()

        fire_gather(j0, r0, gs0)

        def body(jj, carry):
            ja = j0 + 2 * jj
            fire_gather(ja + 1, r1, gs1)
            drain_gather(r0, gs0)
            fire_write(ja, r0, ws0)
            drain_gather(r1, gs1)
            fire_write(ja + 1, r1, ws1)
            drain_write(r0, ws0)

            @pl.when(jj < cpw // 2 - 1)
            def _():
                fire_gather(ja + 2, r0, gs0)

            drain_write(r1, ws1)
            return carry

        lax.fori_loop(0, cpw // 2, body, 0)

    return gather


_gather_q = _make_gather(_QT, 0, _QT)
_gather_d0 = _make_gather(_DT, 0, _DT // 2)
_gather_d1 = _make_gather(_DT, _DT // 2, _DT // 2)


def kernel(query_tok, doc_tok, table):
    q2 = query_tok.T.astype(jnp.int32)
    d2 = doc_tok.T.astype(jnp.int32)
    qf = _gather_q(q2, table)
    d0f = _gather_d0(d2, table)
    d1f = _gather_d1(d2, table)
    q_emb = qf.reshape(_QT, _B, _D).transpose(1, 0, 2)
    d_emb = jnp.concatenate(
        [d0f.reshape(_DT // 2, _B, _D), d1f.reshape(_DT // 2, _B, _D)],
        axis=0).transpose(1, 0, 2)
    return (q_emb, d_emb)
